# baseline (speedup 1.0000x reference)
import functools

import jax
import jax.numpy as jnp
import numpy as np
from jax import lax
from jax.experimental import pallas as pl
from jax.experimental.pallas import tpu as pltpu
from jax.experimental.pallas import tpu_sc as plsc

VOCAB = 100000
D = 128
B = 4096
L = 50
H1 = 100
H2 = 50

NC = 2
NS = 16
NW = NC * NS
NBUF = 5

NSLICE = 2
SB = B // NSLICE


_MESH = plsc.VectorSubcoreMesh(core_axis_name="c", subcore_axis_name="s")


def _make_sc_pool(sb):
    bpw = sb // NW
    rows_total = bpw * L
    rpc = 128 if rows_total % (128 * NBUF) == 0 else 64
    nchunk = rows_total // rpc
    assert nchunk % NBUF == 0 and rpc % 8 == 0

    @functools.partial(
        pl.kernel,
        mesh=_MESH,
        out_type=jax.ShapeDtypeStruct((sb, D), jnp.float32),
        scratch_types=[
            pltpu.VMEM((nchunk, rpc), jnp.int32),
            pltpu.VMEM((nchunk, rpc), jnp.int32),
            pltpu.VMEM((rpc, D), jnp.float32),
            pltpu.VMEM((rpc, D), jnp.float32),
            pltpu.VMEM((rpc, D), jnp.float32),
            pltpu.VMEM((rpc, D), jnp.float32),
            pltpu.VMEM((rpc, D), jnp.float32),
            pltpu.VMEM_SHARED((NS * bpw, D), jnp.float32),
            pltpu.SemaphoreType.DMA,
            pltpu.SemaphoreType.DMA,
            pltpu.SemaphoreType.DMA,
            pltpu.SemaphoreType.DMA,
            pltpu.SemaphoreType.DMA,
        ],
    )
    def pool(emb_hbm, idx_hbm, slot_hbm, zero_hbm, out_hbm, idx_v, slot_v,
             rows_0, rows_1, rows_2, rows_3, rows_4, acc_sh, sem_0, sem_1,
             sem_2, sem_3, sem_4):
        sid = lax.axis_index("s")
        wid = sid * NC + lax.axis_index("c")

        pltpu.sync_copy(idx_hbm.at[wid], idx_v)
        pltpu.sync_copy(slot_hbm.at[wid], slot_v)
        pltpu.sync_copy(zero_hbm, acc_sh.at[pl.ds(sid * bpw, bpw)])

        bufs = (rows_0, rows_1, rows_2, rows_3, rows_4)
        sems = (sem_0, sem_1, sem_2, sem_3, sem_4)

        for k in range(NBUF):
            pltpu.async_copy(emb_hbm.at[idx_v.at[k]], bufs[k], sems[k])

        @pl.loop(0, nchunk, step=NBUF)
        def _chunk(c):
            for k in range(NBUF):
                pltpu.make_async_copy(
                    emb_hbm.at[idx_v.at[0]], bufs[k], sems[k]).wait()
                pltpu.sync_copy(bufs[k], acc_sh.at[slot_v.at[c + k]],
                                add=True)

                @pl.when(c + k + NBUF < nchunk)
                def _(k=k):
                    pltpu.async_copy(
                        emb_hbm.at[idx_v.at[c + k + NBUF]], bufs[k], sems[k])

        pltpu.sync_copy(acc_sh.at[pl.ds(sid * bpw, bpw)],
                        out_hbm.at[pl.ds(wid * bpw, bpw)])

    return pool, rpc, nchunk, bpw


_POOL, _RPC, _NCHUNK, _BPW = _make_sc_pool(SB)

_BLOC = np.arange(SB * L, dtype=np.int32) // L
_SLOTS = np.asarray(
    ((_BLOC // _BPW) // NC) * _BPW + _BLOC % _BPW, dtype=np.int32
).reshape(NW, _NCHUNK, _RPC)
_ZEROS = np.zeros((_BPW, D), dtype=np.float32)


def _mlp_body(x_ref, w1_ref, b1_ref, w2_ref, b2_ref, o_ref):
    x = x_ref[...] * jnp.float32(1.0 / L)
    h = jnp.dot(x, w1_ref[...], preferred_element_type=jnp.float32)
    h = jnp.maximum(h + b1_ref[...], 0.0)
    logits = jnp.dot(h, w2_ref[...], preferred_element_type=jnp.float32)
    logits = logits + b2_ref[...]
    m = jnp.max(logits, axis=1, keepdims=True)
    s = logits - m
    lse = jnp.log(jnp.sum(jnp.exp(s), axis=1, keepdims=True))
    o_ref[...] = s - lse


_MLP_BLOCK = 1024


def _mlp(pooled, W1, b1, W2, b2):
    sb = pooled.shape[0]
    grid = (sb // _MLP_BLOCK,)
    return pl.pallas_call(
        _mlp_body,
        grid=grid,
        in_specs=[
            pl.BlockSpec((_MLP_BLOCK, D), lambda i: (i, 0)),
            pl.BlockSpec((D, H1), lambda i: (0, 0)),
            pl.BlockSpec((1, H1), lambda i: (0, 0)),
            pl.BlockSpec((H1, H2), lambda i: (0, 0)),
            pl.BlockSpec((1, H2), lambda i: (0, 0)),
        ],
        out_specs=pl.BlockSpec((_MLP_BLOCK, H2), lambda i: (i, 0)),
        out_shape=jax.ShapeDtypeStruct((sb, H2), jnp.float32),
    )(pooled, W1, b1, W2, b2)


def kernel(index_list, emb, W1, b1, W2, b2):
    b1r = b1.reshape(1, H1)
    b2r = b2.reshape(1, H2)
    outs = []
    for si in range(NSLICE):
        idx_si = index_list[si * SB:(si + 1) * SB].astype(jnp.int32).reshape(
            NW, _NCHUNK, _RPC)
        pooled = _POOL(emb, idx_si, _SLOTS, _ZEROS)
        outs.append(_mlp(pooled, W1, b1r, W2, b2r))
    return jnp.concatenate(outs, axis=0)


# speedup vs baseline: 1.1481x; 1.1481x over previous
import functools

import jax
import jax.numpy as jnp
import numpy as np
from jax import lax
from jax.experimental import pallas as pl
from jax.experimental.pallas import tpu as pltpu
from jax.experimental.pallas import tpu_sc as plsc

VOCAB = 100000
D = 128
B = 4096
L = 50
H1 = 100
H2 = 50

NC = 2
NS = 16
NW = NC * NS
BPW = B // NW
NBUF = 4

_MESH = plsc.VectorSubcoreMesh(core_axis_name="c", subcore_axis_name="s")

_SLOTS = np.broadcast_to(
    (np.arange(NW, dtype=np.int32)[:, None] // NC * BPW
     + np.arange(BPW, dtype=np.int32)[None, :])[:, :, None],
    (NW, BPW, L)).copy()
_ZEROS = np.zeros((BPW, D), dtype=np.float32)


@functools.partial(
    pl.kernel,
    mesh=_MESH,
    out_type=jax.ShapeDtypeStruct((B, D), jnp.float32),
    compiler_params=pltpu.CompilerParams(use_tc_tiling_on_sc=True),
    scratch_types=[
        pltpu.VMEM((BPW, L), jnp.int32),
        pltpu.VMEM((BPW, L), jnp.int32),
        pltpu.VMEM((L, D), jnp.float32),
        pltpu.VMEM((L, D), jnp.float32),
        pltpu.VMEM((L, D), jnp.float32),
        pltpu.VMEM((L, D), jnp.float32),
        pltpu.VMEM_SHARED((NS * BPW, D), jnp.float32),
        pltpu.SemaphoreType.DMA,
        pltpu.SemaphoreType.DMA,
        pltpu.SemaphoreType.DMA,
        pltpu.SemaphoreType.DMA,
    ],
)
def _sc_pool(emb_hbm, idx_hbm, slot_hbm, zero_hbm, out_hbm, idx_v, slot_v,
             rows_0, rows_1, rows_2, rows_3, acc_sh, sem_0, sem_1, sem_2,
             sem_3):
    sid = lax.axis_index("s")
    wid = sid * NC + lax.axis_index("c")

    pltpu.sync_copy(idx_hbm.at[pl.ds(wid * BPW, BPW)], idx_v)
    pltpu.sync_copy(slot_hbm.at[wid], slot_v)
    pltpu.sync_copy(zero_hbm, acc_sh.at[pl.ds(sid * BPW, BPW)])

    bufs = (rows_0, rows_1, rows_2, rows_3)
    sems = (sem_0, sem_1, sem_2, sem_3)

    for k in range(NBUF):
        pltpu.async_copy(emb_hbm.at[idx_v.at[k]], bufs[k], sems[k])

    @pl.loop(0, BPW, step=NBUF)
    def _row(r):
        for k in range(NBUF):
            pltpu.make_async_copy(
                emb_hbm.at[idx_v.at[0]], bufs[k], sems[k]).wait()
            pltpu.sync_copy(bufs[k], acc_sh.at[slot_v.at[r + k]], add=True)

            @pl.when(r + k + NBUF < BPW)
            def _(k=k):
                pltpu.async_copy(
                    emb_hbm.at[idx_v.at[r + k + NBUF]], bufs[k], sems[k])

    pltpu.sync_copy(acc_sh.at[pl.ds(sid * BPW, BPW)],
                    out_hbm.at[pl.ds(wid * BPW, BPW)])


def _mlp_body(x_ref, w1_ref, b1_ref, w2_ref, b2_ref, o_ref):
    x = x_ref[...] * jnp.float32(1.0 / L)
    h = jnp.dot(x, w1_ref[...], preferred_element_type=jnp.float32)
    h = jnp.maximum(h + b1_ref[...], 0.0)
    logits = jnp.dot(h, w2_ref[...], preferred_element_type=jnp.float32)
    logits = logits + b2_ref[...]
    m = jnp.max(logits, axis=1, keepdims=True)
    s = logits - m
    lse = jnp.log(jnp.sum(jnp.exp(s), axis=1, keepdims=True))
    o_ref[...] = s - lse


_MLP_BLOCK = 1024


def _mlp(pooled, W1, b1, W2, b2):
    sb = pooled.shape[0]
    grid = (sb // _MLP_BLOCK,)
    return pl.pallas_call(
        _mlp_body,
        grid=grid,
        in_specs=[
            pl.BlockSpec((_MLP_BLOCK, D), lambda i: (i, 0)),
            pl.BlockSpec((D, H1), lambda i: (0, 0)),
            pl.BlockSpec((1, H1), lambda i: (0, 0)),
            pl.BlockSpec((H1, H2), lambda i: (0, 0)),
            pl.BlockSpec((1, H2), lambda i: (0, 0)),
        ],
        out_specs=pl.BlockSpec((_MLP_BLOCK, H2), lambda i: (i, 0)),
        out_shape=jax.ShapeDtypeStruct((sb, H2), jnp.float32),
    )(pooled, W1, b1, W2, b2)


def kernel(index_list, emb, W1, b1, W2, b2):
    pooled = _sc_pool(emb, index_list.astype(jnp.int32), _SLOTS, _ZEROS)
    return _mlp(pooled, W1, b1.reshape(1, H1), W2, b2.reshape(1, H2))


# speedup vs baseline: 1.1644x; 1.0142x over previous
import functools

import jax
import jax.numpy as jnp
import numpy as np
from jax import lax
from jax.experimental import pallas as pl
from jax.experimental.pallas import tpu as pltpu
from jax.experimental.pallas import tpu_sc as plsc

VOCAB = 100000
D = 128
B = 4096
L = 50
H1 = 100
H2 = 50

NC = 2
NS = 16
NW = NC * NS
BPW = B // NW
NBUF = 8

_MESH = plsc.VectorSubcoreMesh(core_axis_name="c", subcore_axis_name="s")

_SLOTS = np.broadcast_to(
    (np.arange(NW, dtype=np.int32)[:, None] // NC * BPW
     + np.arange(BPW, dtype=np.int32)[None, :])[:, :, None],
    (NW, BPW, L)).copy()
_ZEROS = np.zeros((BPW, D), dtype=np.float32)


@functools.partial(
    pl.kernel,
    mesh=_MESH,
    out_type=jax.ShapeDtypeStruct((B, D), jnp.float32),
    compiler_params=pltpu.CompilerParams(use_tc_tiling_on_sc=True),
    scratch_types=[
        pltpu.VMEM((BPW, L), jnp.int32),
        pltpu.VMEM((BPW, L), jnp.int32),
        pltpu.VMEM((L, D), jnp.float32),
        pltpu.VMEM((L, D), jnp.float32),
        pltpu.VMEM((L, D), jnp.float32),
        pltpu.VMEM((L, D), jnp.float32),
        pltpu.VMEM((L, D), jnp.float32),
        pltpu.VMEM((L, D), jnp.float32),
        pltpu.VMEM((L, D), jnp.float32),
        pltpu.VMEM((L, D), jnp.float32),
        pltpu.VMEM_SHARED((NS * BPW, D), jnp.float32),
        pltpu.SemaphoreType.DMA,
        pltpu.SemaphoreType.DMA,
        pltpu.SemaphoreType.DMA,
        pltpu.SemaphoreType.DMA,
        pltpu.SemaphoreType.DMA,
        pltpu.SemaphoreType.DMA,
        pltpu.SemaphoreType.DMA,
        pltpu.SemaphoreType.DMA,
    ],
)
def _sc_pool(emb_hbm, idx_hbm, slot_hbm, zero_hbm, out_hbm, idx_v, slot_v,
             rows_0, rows_1, rows_2, rows_3, rows_4, rows_5, rows_6, rows_7,
             acc_sh, sem_0, sem_1, sem_2, sem_3, sem_4, sem_5, sem_6, sem_7):
    sid = lax.axis_index("s")
    wid = sid * NC + lax.axis_index("c")

    pltpu.sync_copy(idx_hbm.at[pl.ds(wid * BPW, BPW)], idx_v)
    pltpu.sync_copy(slot_hbm.at[wid], slot_v)
    pltpu.sync_copy(zero_hbm, acc_sh.at[pl.ds(sid * BPW, BPW)])

    bufs = (rows_0, rows_1, rows_2, rows_3, rows_4, rows_5, rows_6, rows_7)
    sems = (sem_0, sem_1, sem_2, sem_3, sem_4, sem_5, sem_6, sem_7)

    for k in range(NBUF):
        pltpu.async_copy(emb_hbm.at[idx_v.at[k]], bufs[k], sems[k])

    @pl.loop(0, BPW, step=NBUF)
    def _row(r):
        for k in range(NBUF):
            pltpu.make_async_copy(
                emb_hbm.at[idx_v.at[0]], bufs[k], sems[k]).wait()
            pltpu.sync_copy(bufs[k], acc_sh.at[slot_v.at[r + k]], add=True)

            @pl.when(r + k + NBUF < BPW)
            def _(k=k):
                pltpu.async_copy(
                    emb_hbm.at[idx_v.at[r + k + NBUF]], bufs[k], sems[k])

    pltpu.sync_copy(acc_sh.at[pl.ds(sid * BPW, BPW)],
                    out_hbm.at[pl.ds(wid * BPW, BPW)])


def _mlp_body(x_ref, w1_ref, b1_ref, w2_ref, b2_ref, o_ref):
    x = x_ref[...] * jnp.float32(1.0 / L)
    h = jnp.dot(x, w1_ref[...], preferred_element_type=jnp.float32)
    h = jnp.maximum(h + b1_ref[...], 0.0)
    logits = jnp.dot(h, w2_ref[...], preferred_element_type=jnp.float32)
    logits = logits + b2_ref[...]
    m = jnp.max(logits, axis=1, keepdims=True)
    s = logits - m
    lse = jnp.log(jnp.sum(jnp.exp(s), axis=1, keepdims=True))
    o_ref[...] = s - lse


_MLP_BLOCK = 1024


def _mlp(pooled, W1, b1, W2, b2):
    sb = pooled.shape[0]
    grid = (sb // _MLP_BLOCK,)
    return pl.pallas_call(
        _mlp_body,
        grid=grid,
        in_specs=[
            pl.BlockSpec((_MLP_BLOCK, D), lambda i: (i, 0)),
            pl.BlockSpec((D, H1), lambda i: (0, 0)),
            pl.BlockSpec((1, H1), lambda i: (0, 0)),
            pl.BlockSpec((H1, H2), lambda i: (0, 0)),
            pl.BlockSpec((1, H2), lambda i: (0, 0)),
        ],
        out_specs=pl.BlockSpec((_MLP_BLOCK, H2), lambda i: (i, 0)),
        out_shape=jax.ShapeDtypeStruct((sb, H2), jnp.float32),
    )(pooled, W1, b1, W2, b2)


def kernel(index_list, emb, W1, b1, W2, b2):
    pooled = _sc_pool(emb, index_list.astype(jnp.int32), _SLOTS, _ZEROS)
    return _mlp(pooled, W1, b1.reshape(1, H1), W2, b2.reshape(1, H2))


# speedup vs baseline: 1.1765x; 1.0104x over previous
import functools

import jax
import jax.numpy as jnp
import numpy as np
from jax import lax
from jax.experimental import pallas as pl
from jax.experimental.pallas import tpu as pltpu
from jax.experimental.pallas import tpu_sc as plsc

VOCAB = 100000
D = 128
B = 4096
L = 50
H1 = 100
H2 = 50

NC = 2
NS = 16
NW = NC * NS
BPW = B // NW
NBUF = 8

_MESH = plsc.VectorSubcoreMesh(core_axis_name="c", subcore_axis_name="s")

_SLOTS = np.broadcast_to(
    (np.arange(NW, dtype=np.int32)[:, None] // NC * BPW
     + np.arange(BPW, dtype=np.int32)[None, :])[:, :, None],
    (NW, BPW, L)).copy()
_ZEROS = np.zeros((BPW, D), dtype=np.float32)


@functools.partial(
    pl.kernel,
    mesh=_MESH,
    out_type=jax.ShapeDtypeStruct((B, D), jnp.float32),
    compiler_params=pltpu.CompilerParams(use_tc_tiling_on_sc=True),
    scratch_types=[
        pltpu.VMEM((BPW, L), jnp.int32),
        pltpu.VMEM((BPW, L), jnp.int32),
        pltpu.VMEM((L, D), jnp.float32),
        pltpu.VMEM((L, D), jnp.float32),
        pltpu.VMEM((L, D), jnp.float32),
        pltpu.VMEM((L, D), jnp.float32),
        pltpu.VMEM((L, D), jnp.float32),
        pltpu.VMEM((L, D), jnp.float32),
        pltpu.VMEM((L, D), jnp.float32),
        pltpu.VMEM((L, D), jnp.float32),
        pltpu.VMEM_SHARED((NS * BPW, D), jnp.float32),
        pltpu.SemaphoreType.DMA,
        pltpu.SemaphoreType.DMA,
        pltpu.SemaphoreType.DMA,
        pltpu.SemaphoreType.DMA,
        pltpu.SemaphoreType.DMA,
        pltpu.SemaphoreType.DMA,
        pltpu.SemaphoreType.DMA,
        pltpu.SemaphoreType.DMA,
    ],
)
def _sc_pool(emb_hbm, idx_hbm, slot_hbm, zero_hbm, out_hbm, idx_v, slot_v,
             rows_0, rows_1, rows_2, rows_3, rows_4, rows_5, rows_6, rows_7,
             acc_sh, sem_0, sem_1, sem_2, sem_3, sem_4, sem_5, sem_6, sem_7):
    sid = lax.axis_index("s")
    wid = sid * NC + lax.axis_index("c")

    pltpu.sync_copy(idx_hbm.at[pl.ds(wid * BPW, BPW)], idx_v)

    bufs = (rows_0, rows_1, rows_2, rows_3, rows_4, rows_5, rows_6, rows_7)
    sems = (sem_0, sem_1, sem_2, sem_3, sem_4, sem_5, sem_6, sem_7)

    for k in range(NBUF):
        pltpu.async_copy(emb_hbm.at[idx_v.at[k]], bufs[k], sems[k])

    pltpu.sync_copy(slot_hbm.at[wid], slot_v)
    pltpu.sync_copy(zero_hbm, acc_sh.at[pl.ds(sid * BPW, BPW)])

    @pl.loop(0, BPW, step=NBUF)
    def _row(r):
        for k in range(NBUF):
            pltpu.make_async_copy(
                emb_hbm.at[idx_v.at[0]], bufs[k], sems[k]).wait()
            pltpu.sync_copy(bufs[k], acc_sh.at[slot_v.at[r + k]], add=True)

            @pl.when(r + k + NBUF < BPW)
            def _(k=k):
                pltpu.async_copy(
                    emb_hbm.at[idx_v.at[r + k + NBUF]], bufs[k], sems[k])

    pltpu.sync_copy(acc_sh.at[pl.ds(sid * BPW, BPW)],
                    out_hbm.at[pl.ds(wid * BPW, BPW)])


def _mlp_body(x_ref, w1_ref, b1_ref, w2_ref, b2_ref, o_ref):
    x = x_ref[...] * jnp.float32(1.0 / L)
    h = jnp.dot(x, w1_ref[...], preferred_element_type=jnp.float32)
    h = jnp.maximum(h + b1_ref[...], 0.0)
    logits = jnp.dot(h, w2_ref[...], preferred_element_type=jnp.float32)
    logits = logits + b2_ref[...]
    m = jnp.max(logits, axis=1, keepdims=True)
    s = logits - m
    lse = jnp.log(jnp.sum(jnp.exp(s), axis=1, keepdims=True))
    o_ref[...] = s - lse


_MLP_BLOCK = 1024


def _mlp(pooled, W1, b1, W2, b2):
    sb = pooled.shape[0]
    grid = (sb // _MLP_BLOCK,)
    return pl.pallas_call(
        _mlp_body,
        grid=grid,
        in_specs=[
            pl.BlockSpec((_MLP_BLOCK, D), lambda i: (i, 0)),
            pl.BlockSpec((D, H1), lambda i: (0, 0)),
            pl.BlockSpec((1, H1), lambda i: (0, 0)),
            pl.BlockSpec((H1, H2), lambda i: (0, 0)),
            pl.BlockSpec((1, H2), lambda i: (0, 0)),
        ],
        out_specs=pl.BlockSpec((_MLP_BLOCK, H2), lambda i: (i, 0)),
        out_shape=jax.ShapeDtypeStruct((sb, H2), jnp.float32),
    )(pooled, W1, b1, W2, b2)


def kernel(index_list, emb, W1, b1, W2, b2):
    pooled = _sc_pool(emb, index_list.astype(jnp.int32), _SLOTS, _ZEROS)
    return _mlp(pooled, W1, b1.reshape(1, H1), W2, b2.reshape(1, H2))
